# baseline (speedup 1.0000x reference)
import functools

import jax
import jax.numpy as jnp
from jax import lax
from jax.experimental import pallas as pl
from jax.experimental.pallas import tpu as pltpu
from jax.experimental.pallas import tpu_sc as plsc

B, C, H, W = 8, 96, 384, 384
PD = 17
RAD = PD // 2
L = 16
NC, NS = 2, 16
NW = NC * NS

SEG_TOTAL = B * C * PD
SEG_W = SEG_TOTAL // NW
SEG_PAD = 416
CHUNKS_W = 2 * SEG_PAD
IDX_CH = 104
N_DESC = CHUNKS_W // IDX_CH
OUT_W = SEG_W * PD
OUT_PAD = 6944
NROWS = (B * C * H * W) // L

_mesh = plsc.VectorSubcoreMesh(core_axis_name="c", subcore_axis_name="s")


@functools.partial(
    pl.kernel,
    out_type=jax.ShapeDtypeStruct((SEG_TOTAL * PD,), jnp.float32),
    mesh=_mesh,
    scratch_types=[
        pltpu.VMEM((L,), jnp.int32),
        pltpu.VMEM((N_DESC, IDX_CH), jnp.int32),
        pltpu.VMEM((SEG_PAD,), jnp.int32),
        pltpu.VMEM((N_DESC * IDX_CH, L), jnp.float32),
        pltpu.VMEM((OUT_PAD,), jnp.float32),
        pltpu.SemaphoreType.DMA,
    ],
    compiler_params=pltpu.CompilerParams(
        needs_layout_passes=False, use_tc_tiling_on_sc=False),
)
def _patch_sampler(x_hbm, centers_hbm, out_hbm,
                   centers_v, idx_v, p_v, rows_v, out_v, sem):
    wid = lax.axis_index("s") * NC + lax.axis_index("c")
    j0 = wid * SEG_W

    pltpu.sync_copy(centers_hbm, centers_v)

    def phase1(t, carry):
        lanes = lax.iota(jnp.int32, L)
        kc = t * L + lanes
        j = jnp.minimum(j0 + kc, SEG_TOTAL - 1)
        b = j // (C * PD)
        rm = j - b * (C * PD)
        c = rm // PD
        r = rm - c * PD
        cx = plsc.load_gather(centers_v, [2 * b])
        cy = plsc.load_gather(centers_v, [2 * b + 1])
        off = ((b * C + c) * H + cy - RAD + r) * W + cx - RAD
        ch = off >> 4
        g0 = 2 * kc
        g1 = g0 + 1
        plsc.store_scatter(idx_v, [g0 // IDX_CH, g0 % IDX_CH], ch)
        plsc.store_scatter(idx_v, [g1 // IDX_CH, g1 % IDX_CH], ch + 1)
        plsc.store_scatter(p_v, [kc], off & 15)
        return carry

    lax.fori_loop(0, SEG_PAD // L, phase1, 0)
    plsc.subcore_barrier()

    copies = [
        pltpu.async_copy(x_hbm.at[idx_v.at[i]],
                         rows_v.at[pl.ds(i * IDX_CH, IDX_CH)], sem)
        for i in range(N_DESC)
    ]
    for cp in copies:
        cp.wait()

    def phase3(u, carry):
        lanes = lax.iota(jnp.int32, L)
        q = u * L + lanes
        k = q // PD
        r = q - k * PD
        pk = plsc.load_gather(p_v, [k])
        src = k * 32 + pk + r
        val = plsc.load_gather(rows_v, [src // L, src & (L - 1)])
        out_v[pl.ds(u * L, L)] = val
        return carry

    lax.fori_loop(0, OUT_PAD // L, phase3, 0)

    pltpu.sync_copy(out_v.at[pl.ds(0, OUT_W)],
                    out_hbm.at[pl.ds(wid * OUT_W, OUT_W)])


def kernel(bchw, patch_centers, patch_diameter):
    del patch_diameter
    x = bchw.reshape(NROWS, L)
    centers = patch_centers.astype(jnp.int32).reshape(L)
    out = _patch_sampler(x, centers)
    return out.reshape(B, C, PD, PD)


# speedup vs baseline: 7.3966x; 7.3966x over previous
import functools

import jax
import jax.numpy as jnp
from jax import lax
from jax.experimental import pallas as pl
from jax.experimental import pallas as pl
from jax.experimental.pallas import tpu as pltpu
from jax.experimental.pallas import tpu_sc as plsc

B, C, H, W = 8, 96, 384, 384
PD = 17
RAD = PD // 2
L = 16
NC, NS = 2, 16
NW = NC * NS

PAIRS = B * C
PAIRS_W = PAIRS // NW
PP = PD * PD
OUT_W = PAIRS_W * PP
OUT_PAD = 6960
SLAB_ROWS = 24
INNER = (PP + L - 1) // L

_mesh = plsc.VectorSubcoreMesh(core_axis_name="c", subcore_axis_name="s")


@functools.partial(
    pl.kernel,
    out_type=jax.ShapeDtypeStruct((PAIRS * PP,), jnp.float32),
    mesh=_mesh,
    scratch_types=[
        pltpu.VMEM((L,), jnp.int32),
        pltpu.VMEM((SLAB_ROWS, W), jnp.float32),
        pltpu.VMEM((OUT_PAD,), jnp.float32),
    ],
    compiler_params=pltpu.CompilerParams(needs_layout_passes=False),
)
def _patch_sampler(x_hbm, centers_hbm, out_hbm,
                   centers_v, stage_v, out_v):
    wid = lax.axis_index("s") * NC + lax.axis_index("c")

    pltpu.sync_copy(centers_hbm, centers_v)

    def pair_body(m, carry):
        lanes = lax.iota(jnp.int32, L)
        cvec = centers_v[...]
        p_glob = wid * PAIRS_W + m
        b = p_glob // C
        c = p_glob - b * C
        cx = jnp.max(jnp.where(lanes == 2 * b, cvec, 0))
        cy = jnp.max(jnp.where(lanes == 2 * b + 1, cvec, 0))
        top = cy - RAD
        left = cx - RAD
        top_a = pl.multiple_of(top & ~7, 8)
        pltpu.sync_copy(x_hbm.at[b, c, pl.ds(top_a, SLAB_ROWS), :], stage_v)
        dr = top - top_a

        def extract(u, cc):
            lanes_i = lax.iota(jnp.int32, L)
            q = jnp.minimum(u * L + lanes_i, PP - 1)
            r = q // PD
            ci = q - r * PD
            val = plsc.load_gather(stage_v, [dr + r, left + ci])
            out_v[pl.ds(m * PP + u * L, L)] = val
            return cc

        lax.fori_loop(0, INNER, extract, 0)
        return carry

    lax.fori_loop(0, PAIRS_W, pair_body, 0)

    pltpu.sync_copy(out_v.at[pl.ds(0, OUT_W)],
                    out_hbm.at[pl.ds(wid * OUT_W, OUT_W)])


def kernel(bchw, patch_centers, patch_diameter):
    del patch_diameter
    centers = patch_centers.astype(jnp.int32).reshape(L)
    out = _patch_sampler(bchw, centers)
    return out.reshape(B, C, PD, PD)


# speedup vs baseline: 8.6464x; 1.1690x over previous
import functools

import jax
import jax.numpy as jnp
from jax import lax
from jax.experimental import pallas as pl
from jax.experimental import pallas as pl
from jax.experimental.pallas import tpu as pltpu
from jax.experimental.pallas import tpu_sc as plsc

B, C, H, W = 8, 96, 384, 384
PD = 17
RAD = PD // 2
L = 16
NC, NS = 2, 16
NW = NC * NS

PAIRS = B * C
PAIRS_W = PAIRS // NW
PP = PD * PD
OUT_W = PAIRS_W * PP
OUT_PAD = 6960
SLAB_ROWS = 24
SLAB_COLS = 256
INNER = (PP + L - 1) // L

_mesh = plsc.VectorSubcoreMesh(core_axis_name="c", subcore_axis_name="s")


@functools.partial(
    pl.kernel,
    out_type=jax.ShapeDtypeStruct((PAIRS * PP,), jnp.float32),
    mesh=_mesh,
    scratch_types=[
        pltpu.VMEM((L,), jnp.int32),
        pltpu.VMEM((2, SLAB_ROWS, SLAB_COLS), jnp.float32),
        pltpu.VMEM((OUT_PAD,), jnp.float32),
        pltpu.SemaphoreType.DMA,
        pltpu.SemaphoreType.DMA,
    ],
    compiler_params=pltpu.CompilerParams(needs_layout_passes=False),
)
def _patch_sampler(x_hbm, centers_hbm, out_hbm,
                   centers_v, stage_v, out_v, sem0, sem1):
    wid = lax.axis_index("s") * NC + lax.axis_index("c")

    pltpu.sync_copy(centers_hbm, centers_v)
    sems = (sem0, sem1)

    def start_copy(m):
        lanes = lax.iota(jnp.int32, L)
        cvec = centers_v[...]
        p_glob = wid * PAIRS_W + m
        b = p_glob // C
        c = p_glob - b * C
        cx = jnp.max(jnp.where(lanes == 2 * b, cvec, 0))
        cy = jnp.max(jnp.where(lanes == 2 * b + 1, cvec, 0))
        top = cy - RAD
        left = cx - RAD
        top_a = pl.multiple_of(top & ~7, 8)
        left_a = pl.multiple_of(jnp.minimum(left & ~127, W - SLAB_COLS), 128)
        cp = pltpu.async_copy(
            x_hbm.at[b, c, pl.ds(top_a, SLAB_ROWS), pl.ds(left_a, SLAB_COLS)],
            stage_v.at[m % 2], sems[m % 2])
        return cp, top - top_a, left - left_a

    cp, dr, dc = start_copy(0)
    for m in range(PAIRS_W):
        if m + 1 < PAIRS_W:
            cp_n, dr_n, dc_n = start_copy(m + 1)
        cp.wait()

        def extract(u, cc, m=m, dr=dr, dc=dc):
            lanes_i = lax.iota(jnp.int32, L)
            q = jnp.minimum(u * L + lanes_i, PP - 1)
            r = q // PD
            ci = q - r * PD
            val = plsc.load_gather(stage_v.at[m % 2], [dr + r, dc + ci])
            out_v[pl.ds(m * PP + u * L, L)] = val
            return cc

        lax.fori_loop(0, INNER, extract, 0)
        if m + 1 < PAIRS_W:
            cp, dr, dc = cp_n, dr_n, dc_n

    pltpu.sync_copy(out_v.at[pl.ds(0, OUT_W)],
                    out_hbm.at[pl.ds(wid * OUT_W, OUT_W)])


def kernel(bchw, patch_centers, patch_diameter):
    del patch_diameter
    centers = patch_centers.astype(jnp.int32).reshape(L)
    out = _patch_sampler(bchw, centers)
    return out.reshape(B, C, PD, PD)


# speedup vs baseline: 8.6789x; 1.0038x over previous
import functools

import jax
import jax.numpy as jnp
import numpy as np
from jax import lax
from jax.experimental import pallas as pl
from jax.experimental.pallas import tpu as pltpu
from jax.experimental.pallas import tpu_sc as plsc

B, C, H, W = 8, 96, 384, 384
PD = 17
RAD = PD // 2
L = 16
NC, NS = 2, 16
NW = NC * NS

PAIRS = B * C
PAIRS_W = PAIRS // NW
PP = PD * PD
OUT_W = PAIRS_W * PP
OUT_PAD = 6976
SLAB_ROWS = 24
SLAB_COLS = 256
INNER = (PP + L - 1) // L
NBUF = 3


_mesh = plsc.VectorSubcoreMesh(core_axis_name="c", subcore_axis_name="s")


@functools.partial(
    pl.kernel,
    out_type=jax.ShapeDtypeStruct((PAIRS * PP,), jnp.float32),
    mesh=_mesh,
    scratch_types=[
        pltpu.VMEM((L,), jnp.int32),
        pltpu.VMEM((NBUF, SLAB_ROWS, SLAB_COLS), jnp.float32),
        pltpu.VMEM((OUT_PAD,), jnp.float32),
        pltpu.VMEM((INNER * L,), jnp.int32),
        pltpu.VMEM((INNER * L,), jnp.int32),
    ] + [pltpu.SemaphoreType.DMA] * NBUF,
    compiler_params=pltpu.CompilerParams(needs_layout_passes=False),
)
def _patch_sampler(x_hbm, centers_hbm, out_hbm,
                   centers_v, stage_v, out_v, rtab_v, ctab_v, *sems):
    wid = lax.axis_index("s") * NC + lax.axis_index("c")

    pltpu.sync_copy(centers_hbm, centers_v)
    lanes = lax.iota(jnp.int32, L)
    cvec = centers_v[...]

    def fill_tabs(u, carry):
        q = jnp.minimum(u * L + lanes, PP - 1)
        r = q // PD
        rtab_v[pl.ds(u * L, L)] = r
        ctab_v[pl.ds(u * L, L)] = q - r * PD
        return carry

    lax.fori_loop(0, INNER, fill_tabs, 0)

    def start_copy(m):
        p_glob = wid * PAIRS_W + m
        b = p_glob // C
        c = p_glob - b * C
        cx = jnp.max(jnp.where(lanes == 2 * b, cvec, 0))
        cy = jnp.max(jnp.where(lanes == 2 * b + 1, cvec, 0))
        top = cy - RAD
        left = cx - RAD
        top_a = pl.multiple_of(top & ~7, 8)
        left_a = pl.multiple_of(jnp.minimum(left & ~127, W - SLAB_COLS), 128)
        cp = pltpu.async_copy(
            x_hbm.at[b, c, pl.ds(top_a, SLAB_ROWS), pl.ds(left_a, SLAB_COLS)],
            stage_v.at[m % NBUF], sems[m % NBUF])
        return cp, top - top_a, left - left_a

    pend = [start_copy(m) for m in range(NBUF - 1)]
    for m in range(PAIRS_W):
        if m + NBUF - 1 < PAIRS_W:
            pend.append(start_copy(m + NBUF - 1))
        cp, dr, dc = pend.pop(0)
        cp.wait()
        slab = stage_v.at[m % NBUF]
        for u in range(INNER):
            row = dr + rtab_v[pl.ds(u * L, L)]
            col = dc + ctab_v[pl.ds(u * L, L)]
            val = plsc.load_gather(slab, [row, col])
            out_v[pl.ds(m * PP + u * L, L)] = val

    pltpu.sync_copy(out_v.at[pl.ds(0, OUT_W)],
                    out_hbm.at[pl.ds(wid * OUT_W, OUT_W)])


def kernel(bchw, patch_centers, patch_diameter):
    del patch_diameter
    centers = patch_centers.astype(jnp.int32).reshape(L)
    out = _patch_sampler(bchw, centers)
    return out.reshape(B, C, PD, PD)


# speedup vs baseline: 8.9978x; 1.0368x over previous
import functools

import jax
import jax.numpy as jnp
from jax import lax
from jax.experimental import pallas as pl
from jax.experimental.pallas import tpu as pltpu
from jax.experimental.pallas import tpu_sc as plsc

B, C, H, W = 8, 96, 384, 384
PD = 17
RAD = PD // 2
L = 16
NC, NS = 2, 16
NW = NC * NS

PAIRS = B * C
PAIRS_W = PAIRS // NW
PP = PD * PD
SLAB_ROWS = 24
SLAB_COLS = 256
INNER = (PP + L - 1) // L
NBUF = 3
NOB = 2

_mesh = plsc.VectorSubcoreMesh(core_axis_name="c", subcore_axis_name="s")


@functools.partial(
    pl.kernel,
    out_type=jax.ShapeDtypeStruct((B, C, PD, PD), jnp.float32),
    mesh=_mesh,
    scratch_types=[
        pltpu.VMEM((L,), jnp.int32),
        pltpu.VMEM((NBUF, SLAB_ROWS, SLAB_COLS), jnp.float32),
        pltpu.VMEM((NOB, PD, PD), jnp.float32),
        pltpu.VMEM((INNER * L,), jnp.int32),
        pltpu.VMEM((INNER * L,), jnp.int32),
    ] + [pltpu.SemaphoreType.DMA] * (NBUF + NOB),
    compiler_params=pltpu.CompilerParams(needs_layout_passes=False),
)
def _patch_sampler(x_hbm, centers_hbm, out_hbm,
                   centers_v, stage_v, pbuf_v, rtab_v, ctab_v, *sems):
    wid = lax.axis_index("s") * NC + lax.axis_index("c")

    pltpu.sync_copy(centers_hbm, centers_v)
    lanes = lax.iota(jnp.int32, L)
    cvec = centers_v[...]

    def fill_tabs(u, carry):
        q = jnp.minimum(u * L + lanes, PP - 1)
        r = q // PD
        rtab_v[pl.ds(u * L, L)] = r
        ctab_v[pl.ds(u * L, L)] = q - r * PD
        return carry

    lax.fori_loop(0, INNER, fill_tabs, 0)

    def bc(m):
        p_glob = wid * PAIRS_W + m
        b = p_glob // C
        return b, p_glob - b * C

    def start_copy(m):
        b, c = bc(m)
        cx = jnp.max(jnp.where(lanes == 2 * b, cvec, 0))
        cy = jnp.max(jnp.where(lanes == 2 * b + 1, cvec, 0))
        top = cy - RAD
        left = cx - RAD
        top_a = pl.multiple_of(top & ~7, 8)
        left_a = pl.multiple_of(jnp.minimum(left & ~127, W - SLAB_COLS), 128)
        cp = pltpu.async_copy(
            x_hbm.at[b, c, pl.ds(top_a, SLAB_ROWS), pl.ds(left_a, SLAB_COLS)],
            stage_v.at[m % NBUF], sems[m % NBUF])
        return cp, top - top_a, left - left_a

    pend = [start_copy(m) for m in range(NBUF - 1)]
    out_pend = []
    for m in range(PAIRS_W):
        if m + NBUF - 1 < PAIRS_W:
            pend.append(start_copy(m + NBUF - 1))
        cp, dr, dc = pend.pop(0)
        cp.wait()
        if len(out_pend) == NOB:
            out_pend.pop(0).wait()
        slab = stage_v.at[m % NBUF]
        pbuf = pbuf_v.at[m % NOB]
        for u in range(INNER):
            rr = rtab_v[pl.ds(u * L, L)]
            cc = ctab_v[pl.ds(u * L, L)]
            val = plsc.load_gather(slab, [dr + rr, dc + cc])
            plsc.store_scatter(pbuf, [rr, cc], val)
        b, c = bc(m)
        out_pend.append(
            pltpu.async_copy(pbuf, out_hbm.at[b, c], sems[NBUF + m % NOB]))
    for cp in out_pend:
        cp.wait()


def kernel(bchw, patch_centers, patch_diameter):
    del patch_diameter
    centers = patch_centers.astype(jnp.int32).reshape(L)
    return _patch_sampler(bchw, centers)
